# Initial kernel scaffold; baseline (speedup 1.0000x reference)
import jax
import jax.numpy as jnp
from jax.experimental import pallas as pl


def kernel(voxels_allAtom_jigsaw, voxels_perAA_jigsaw, voxels_allAtom_full, voxels_perAA_full, prot_feats, centerIdx_jigsaw, resIds_jigsaw, centerIdx_full, resIds_full):
    raise NotImplementedError("write your pallas kernel here")


# speedup vs baseline: 5.1502x; 5.1502x over previous
import jax
import jax.numpy as jnp
from jax import lax
from jax.experimental import pallas as pl
from jax.experimental.pallas import tpu as pltpu
from jax.experimental.pallas import tpu_sc as plsc

B = 4
G = 32
NV = B * G * G * G
D = 64
NRES = 2048
KJ = 16384
KF = 32768
NW = 32
OWN = NV // NW
OWN_SHIFT = 12
FLUSH = 128
CHUNK = 8192
ZROW = NRES


def _sc_body(protD, protZL, protZR, cj_hbm, rj_hbm, cf_hbm, rf_hbm, zsrc_hbm,
             gc_hbm,
             cb0, cb1, rb0, rb1, lastj, lastf, wr, wv, wb,
             ridx, vidx, bidx, rowsA, rowsB, sem, sem_ms):
  wid = lax.axis_index("s") * 2 + lax.axis_index("c")
  base = wid * OWN
  iota = lax.iota(jnp.int32, 16)

  pltpu.sync_copy(zsrc_hbm, rowsA)
  memset_descs = [
      pltpu.async_copy(rowsA, gc_hbm.at[pl.ds(base + i * FLUSH, FLUSH)],
                       sem_ms)
      for i in range(OWN // FLUSH)
  ]

  def _clr(i, _):
    z = jnp.zeros((16,), jnp.int32)
    lastj[pl.ds(i * 16, 16)] = z
    lastf[pl.ds(i * 16, 16)] = z
    return 0
  lax.fori_loop(0, OWN // 16, _clr, 0)

  def _scan(c_hbm, r_hbm, k, tab):
    nch = k // CHUNK
    bufs = ((cb0, rb0), (cb1, rb1))

    def launch(ch):
      cb, rb = bufs[ch % 2]
      return (pltpu.async_copy(c_hbm.at[pl.ds(ch * CHUNK, CHUNK)], cb, sem),
              pltpu.async_copy(r_hbm.at[pl.ds(ch * CHUNK, CHUNK)], rb, sem))

    pend = launch(0)
    for ch in range(nch):
      for d in pend:
        d.wait()
      if ch + 1 < nch:
        pend = launch(ch + 1)
      cb, rb = bufs[ch % 2]

      def body(i, _):
        v = cb[pl.ds(i * 16, 16)]
        r = rb[pl.ds(i * 16, 16)]
        own = lax.shift_right_logical(v, OWN_SHIFT) == wid
        loc = v & (OWN - 1)
        plsc.store_scatter(tab, [loc], r + 1, mask=own)
        return 0
      lax.fori_loop(0, CHUNK // 16, body, 0)

  _scan(cj_hbm, rj_hbm, KJ, lastj)
  _scan(cf_hbm, rf_hbm, KF, lastf)

  for d in memset_descs:
    d.wait()

  def _pass(sel, srcA, srcB, two_src):
    def compact(t, carry):
      off, vl, al, bl = carry
      lj = lastj[pl.ds(t * 16, 16)]
      lf = lastf[pl.ds(t * 16, 16)]
      m, ra, rbv = sel(lj, lf)
      vg = base + t * 16 + iota
      mi = m.astype(jnp.int32)
      pos = off + plsc.cumsum(mi) - 1
      plsc.store_scatter(wr, [pos], ra, mask=m)
      plsc.store_scatter(wb, [pos], rbv, mask=m)
      plsc.store_scatter(wv, [pos], vg, mask=m)
      cnt = jnp.sum(mi)
      vmax = jnp.max(jnp.where(m, vg, -1))
      m2 = m & (vg == vmax)
      amax = jnp.max(jnp.where(m2, ra, -1))
      bmax = jnp.max(jnp.where(m2, rbv, -1))
      upd = vmax > vl
      return (off + cnt,
              jnp.where(upd, vmax, vl),
              jnp.where(upd, amax, al),
              jnp.where(upd, bmax, bl))

    n, vl, al, bl = lax.fori_loop(0, OWN // 16, compact, (0, -1, -1, -1))
    npad = ((n + FLUSH - 1) // FLUSH) * FLUSH

    def pad(c, _):
      p = c * 16 + iota
      pm = (p >= n) & (p < npad)
      plsc.store_scatter(wr, [p], jnp.full((16,), al, jnp.int32), mask=pm)
      plsc.store_scatter(wb, [p], jnp.full((16,), bl, jnp.int32), mask=pm)
      plsc.store_scatter(wv, [p], jnp.full((16,), vl, jnp.int32), mask=pm)
      return 0
    lax.fori_loop(lax.shift_right_logical(n, 4),
                  lax.shift_right_logical(npad, 4), pad, 0)

    def flush(f, _):
      def cp(c, _):
        ridx[pl.ds(c * 16, 16)] = wr[pl.ds(f * FLUSH + c * 16, 16)]
        vidx[pl.ds(c * 16, 16)] = wv[pl.ds(f * FLUSH + c * 16, 16)]
        bidx[pl.ds(c * 16, 16)] = wb[pl.ds(f * FLUSH + c * 16, 16)]
        return 0
      lax.fori_loop(0, FLUSH // 16, cp, 0)
      ga = pltpu.async_copy(srcA.at[ridx], rowsA, sem)
      if two_src:
        gb = pltpu.async_copy(srcB.at[bidx], rowsB, sem)
        ga.wait()
        gb.wait()

        def addrow(i, _):
          r = i // (D * 2 // 16)
          c = (i % (D * 2 // 16)) * 16
          rowsA[r, pl.ds(c, 16)] = rowsA[r, pl.ds(c, 16)] + rowsB[r, pl.ds(c, 16)]
          return 0
        lax.fori_loop(0, FLUSH * (D * 2 // 16), addrow, 0)
      else:
        ga.wait()
      pltpu.async_copy(rowsA, gc_hbm.at[vidx], sem).wait()
      return 0
    lax.fori_loop(0, npad // FLUSH, flush, 0)

  zvec = jnp.full((16,), ZROW, jnp.int32)
  _pass(lambda lj, lf: ((lj > 0) & (lf == 0), lj - 1, zvec),
        protD, protZL, False)
  _pass(lambda lj, lf: ((lj == 0) & (lf > 0), lf - 1, zvec),
        protZL, protZL, False)
  _pass(lambda lj, lf: ((lj > 0) & (lf > 0), lj - 1, lf - 1),
        protZR, protZL, True)


@jax.jit
def _sc_scatter(protD, protZL, protZR, cj, rj, cf, rf):
  mesh = plsc.VectorSubcoreMesh(core_axis_name="c", subcore_axis_name="s",
                                num_cores=2, num_subcores=16)
  zsrc = jnp.zeros((FLUSH, 2 * D), jnp.float32)
  f = pl.kernel(
      _sc_body,
      out_type=jax.ShapeDtypeStruct((NV, 2 * D), jnp.float32),
      mesh=mesh,
      compiler_params=pltpu.CompilerParams(needs_layout_passes=False),
      scratch_types=[
          pltpu.VMEM((CHUNK,), jnp.int32),
          pltpu.VMEM((CHUNK,), jnp.int32),
          pltpu.VMEM((CHUNK,), jnp.int32),
          pltpu.VMEM((CHUNK,), jnp.int32),
          pltpu.VMEM((OWN,), jnp.int32),
          pltpu.VMEM((OWN,), jnp.int32),
          pltpu.VMEM((OWN + 16,), jnp.int32),
          pltpu.VMEM((OWN + 16,), jnp.int32),
          pltpu.VMEM((OWN + 16,), jnp.int32),
          pltpu.VMEM((FLUSH,), jnp.int32),
          pltpu.VMEM((FLUSH,), jnp.int32),
          pltpu.VMEM((FLUSH,), jnp.int32),
          pltpu.VMEM((FLUSH, 2 * D), jnp.float32),
          pltpu.VMEM((FLUSH, 2 * D), jnp.float32),
          pltpu.SemaphoreType.DMA,
          pltpu.SemaphoreType.DMA,
      ],
  )
  return f(protD, protZL, protZR, cj, rj, cf, rf, zsrc)


TYZ = 64


def _tc_body(aj, pj, af, pf, gc, oj, of):
  for ref, out, c0 in ((aj, oj, 0), (pj, oj, 4), (af, of, 0), (pf, of, 4)):
    cn = ref.shape[3]
    out[0, c0:c0 + cn] = jnp.transpose(ref[0], (2, 1, 0))
  tr = jnp.transpose(gc[0], (2, 1, 0))
  oj[0, 84:148] = tr[:D]
  of[0, 84:148] = tr[D:]
  z = jnp.zeros((TYZ, G), jnp.float32)
  o = jnp.ones((TYZ, G), jnp.float32)
  oj[0, 148] = z
  oj[0, 149] = o
  of[0, 148] = o
  of[0, 149] = z


@jax.jit
def _tc_fuse(aj, pj, af, pf, gc):
  YZ = G * G
  nc = 150

  def in_spec(c):
    return pl.BlockSpec((1, G, TYZ, c), lambda b, t: (b, 0, t, 0))

  out_spec = pl.BlockSpec((1, nc, TYZ, G), lambda b, t: (b, 0, t, 0))
  return pl.pallas_call(
      _tc_body,
      grid=(B, YZ // TYZ),
      in_specs=[in_spec(4), in_spec(80), in_spec(4), in_spec(80),
                in_spec(2 * D)],
      out_specs=[out_spec, out_spec],
      out_shape=[jax.ShapeDtypeStruct((B, nc, YZ, G), jnp.float32),
                 jax.ShapeDtypeStruct((B, nc, YZ, G), jnp.float32)],
  )(aj, pj, af, pf, gc)


def kernel(voxels_allAtom_jigsaw, voxels_perAA_jigsaw, voxels_allAtom_full,
           voxels_perAA_full, prot_feats, centerIdx_jigsaw, resIds_jigsaw,
           centerIdx_full, resIds_full):
  z = jnp.zeros((NRES + 1, D), jnp.float32)
  pp = z.at[:NRES].set(prot_feats)
  protD = jnp.concatenate([pp, pp], axis=1)
  protZL = jnp.concatenate([jnp.zeros_like(pp), pp], axis=1)
  protZR = jnp.concatenate([pp, jnp.zeros_like(pp)], axis=1)
  gc = _sc_scatter(protD, protZL, protZR,
                   centerIdx_jigsaw.astype(jnp.int32),
                   resIds_jigsaw.astype(jnp.int32),
                   centerIdx_full.astype(jnp.int32),
                   resIds_full.astype(jnp.int32))
  gc4 = gc.reshape(B, G, G * G, 2 * D)
  aj = voxels_allAtom_jigsaw.reshape(B, G, G * G, 4)
  pj = voxels_perAA_jigsaw.reshape(B, G, G * G, 80)
  af = voxels_allAtom_full.reshape(B, G, G * G, 4)
  pf = voxels_perAA_full.reshape(B, G, G * G, 80)
  oj, of = _tc_fuse(aj, pj, af, pf, gc4)
  return (oj.reshape(B, 150, G, G, G), of.reshape(B, 150, G, G, G))


# speedup vs baseline: 12.0626x; 2.3421x over previous
import jax
import jax.numpy as jnp
from jax import lax
from jax.experimental import pallas as pl
from jax.experimental.pallas import tpu as pltpu
from jax.experimental.pallas import tpu_sc as plsc

B = 4
G = 32
NV = B * G * G * G
D = 64
NRES = 2048
KJ = 16384
KF = 32768
NW = 32
OWN = NV // NW
OWN_SHIFT = 12
FLUSH = 128
CHUNK = 8192
ZROW = NRES


def _sc_body(protD, protZL, protZR, cj_hbm, rj_hbm, cf_hbm, rf_hbm, zsrc_hbm,
             gc_hbm,
             cb0, cb1, rb0, rb1, lastj, lastf, wr, wv, wb,
             ridx, vidx, bidx, rowsA, rowsB, sem, sem_ms):
  wid = lax.axis_index("s") * 2 + lax.axis_index("c")
  base = wid * OWN
  iota = lax.iota(jnp.int32, 16)

  pltpu.sync_copy(zsrc_hbm, rowsA)
  memset_descs = [
      pltpu.async_copy(rowsA, gc_hbm.at[pl.ds(base + i * FLUSH, FLUSH)],
                       sem_ms)
      for i in range(OWN // FLUSH)
  ]

  def _clr(i, _):
    z = jnp.zeros((16,), jnp.int32)
    lastj[pl.ds(i * 16, 16)] = z
    lastf[pl.ds(i * 16, 16)] = z
    return 0
  lax.fori_loop(0, OWN // 16, _clr, 0)

  def _scan(c_hbm, r_hbm, k, tab):
    nch = k // CHUNK
    bufs = ((cb0, rb0), (cb1, rb1))

    def launch(ch):
      cb, rb = bufs[ch % 2]
      return (pltpu.async_copy(c_hbm.at[pl.ds(ch * CHUNK, CHUNK)], cb, sem),
              pltpu.async_copy(r_hbm.at[pl.ds(ch * CHUNK, CHUNK)], rb, sem))

    pend = launch(0)
    for ch in range(nch):
      for d in pend:
        d.wait()
      if ch + 1 < nch:
        pend = launch(ch + 1)
      cb, rb = bufs[ch % 2]

      def body(i, _):
        v = cb[pl.ds(i * 16, 16)]
        r = rb[pl.ds(i * 16, 16)]
        own = lax.shift_right_logical(v, OWN_SHIFT) == wid
        loc = v & (OWN - 1)
        plsc.store_scatter(tab, [loc], r + 1, mask=own)
        return 0
      lax.fori_loop(0, CHUNK // 16, body, 0)

  _scan(cj_hbm, rj_hbm, KJ, lastj)
  _scan(cf_hbm, rf_hbm, KF, lastf)

  for d in memset_descs:
    d.wait()

  def _pass(sel, srcA, srcB, two_src):
    def compact(t, carry):
      off, vl, al, bl = carry
      lj = lastj[pl.ds(t * 16, 16)]
      lf = lastf[pl.ds(t * 16, 16)]
      m, ra, rbv = sel(lj, lf)
      vg = base + t * 16 + iota
      mi = m.astype(jnp.int32)
      pos = off + plsc.cumsum(mi) - 1
      plsc.store_scatter(wr, [pos], ra, mask=m)
      plsc.store_scatter(wb, [pos], rbv, mask=m)
      plsc.store_scatter(wv, [pos], vg, mask=m)
      cnt = jnp.sum(mi)
      vmax = jnp.max(jnp.where(m, vg, -1))
      m2 = m & (vg == vmax)
      amax = jnp.max(jnp.where(m2, ra, -1))
      bmax = jnp.max(jnp.where(m2, rbv, -1))
      upd = vmax > vl
      return (off + cnt,
              jnp.where(upd, vmax, vl),
              jnp.where(upd, amax, al),
              jnp.where(upd, bmax, bl))

    n, vl, al, bl = lax.fori_loop(0, OWN // 16, compact, (0, -1, -1, -1))
    npad = ((n + FLUSH - 1) // FLUSH) * FLUSH

    def pad(c, _):
      p = c * 16 + iota
      pm = (p >= n) & (p < npad)
      plsc.store_scatter(wr, [p], jnp.full((16,), al, jnp.int32), mask=pm)
      plsc.store_scatter(wb, [p], jnp.full((16,), bl, jnp.int32), mask=pm)
      plsc.store_scatter(wv, [p], jnp.full((16,), vl, jnp.int32), mask=pm)
      return 0
    lax.fori_loop(lax.shift_right_logical(n, 4),
                  lax.shift_right_logical(npad, 4), pad, 0)

    def flush(f, _):
      def cp(c, _):
        ridx[pl.ds(c * 16, 16)] = wr[pl.ds(f * FLUSH + c * 16, 16)]
        vidx[pl.ds(c * 16, 16)] = wv[pl.ds(f * FLUSH + c * 16, 16)]
        bidx[pl.ds(c * 16, 16)] = wb[pl.ds(f * FLUSH + c * 16, 16)]
        return 0
      lax.fori_loop(0, FLUSH // 16, cp, 0)
      ga = pltpu.async_copy(srcA.at[ridx], rowsA, sem)
      if two_src:
        gb = pltpu.async_copy(srcB.at[bidx], rowsB, sem)
        ga.wait()
        gb.wait()

        def addrow(i, _):
          r = i // (D * 2 // 16)
          c = (i % (D * 2 // 16)) * 16
          rowsA[r, pl.ds(c, 16)] = rowsA[r, pl.ds(c, 16)] + rowsB[r, pl.ds(c, 16)]
          return 0
        lax.fori_loop(0, FLUSH * (D * 2 // 16), addrow, 0)
      else:
        ga.wait()
      pltpu.async_copy(rowsA, gc_hbm.at[vidx], sem).wait()
      return 0
    lax.fori_loop(0, npad // FLUSH, flush, 0)

  zvec = jnp.full((16,), ZROW, jnp.int32)
  _pass(lambda lj, lf: ((lj > 0) & (lf == 0), lj - 1, zvec),
        protD, protZL, False)
  _pass(lambda lj, lf: ((lj == 0) & (lf > 0), lf - 1, zvec),
        protZL, protZL, False)
  _pass(lambda lj, lf: ((lj > 0) & (lf > 0), lj - 1, lf - 1),
        protZR, protZL, True)


@jax.jit
def _sc_scatter(protD, protZL, protZR, cj, rj, cf, rf):
  mesh = plsc.VectorSubcoreMesh(core_axis_name="c", subcore_axis_name="s",
                                num_cores=2, num_subcores=16)
  zsrc = jnp.zeros((FLUSH, 2 * D), jnp.float32)
  f = pl.kernel(
      _sc_body,
      out_type=jax.ShapeDtypeStruct((NV, 2 * D), jnp.float32),
      mesh=mesh,
      compiler_params=pltpu.CompilerParams(needs_layout_passes=False),
      scratch_types=[
          pltpu.VMEM((CHUNK,), jnp.int32),
          pltpu.VMEM((CHUNK,), jnp.int32),
          pltpu.VMEM((CHUNK,), jnp.int32),
          pltpu.VMEM((CHUNK,), jnp.int32),
          pltpu.VMEM((OWN,), jnp.int32),
          pltpu.VMEM((OWN,), jnp.int32),
          pltpu.VMEM((OWN + 16,), jnp.int32),
          pltpu.VMEM((OWN + 16,), jnp.int32),
          pltpu.VMEM((OWN + 16,), jnp.int32),
          pltpu.VMEM((FLUSH,), jnp.int32),
          pltpu.VMEM((FLUSH,), jnp.int32),
          pltpu.VMEM((FLUSH,), jnp.int32),
          pltpu.VMEM((FLUSH, 2 * D), jnp.float32),
          pltpu.VMEM((FLUSH, 2 * D), jnp.float32),
          pltpu.SemaphoreType.DMA,
          pltpu.SemaphoreType.DMA,
      ],
  )
  return f(protD, protZL, protZR, cj, rj, cf, rf, zsrc)


TYZ = 512
XB = 8


def _tc_body(aj, pj, af, pf, gc, oj, of):
  z = jnp.zeros((XB, TYZ, 1), jnp.float32)
  o = jnp.ones((XB, TYZ, 1), jnp.float32)
  g = gc[0]
  catj = jnp.concatenate([aj[0], pj[0], g[:, :, :D], z, o], axis=-1)
  catf = jnp.concatenate([af[0], pf[0], g[:, :, D:], o, z], axis=-1)
  oj[0] = jnp.transpose(catj, (1, 0, 2))
  of[0] = jnp.transpose(catf, (1, 0, 2))


@jax.jit
def _tc_fuse(aj, pj, af, pf, gc):
  YZ = G * G
  nc = 150

  def in_spec(c):
    return pl.BlockSpec((1, XB, TYZ, c), lambda b, x, t: (b, x, t, 0))

  out_spec = pl.BlockSpec((1, TYZ, XB, nc), lambda b, x, t: (b, t, x, 0))
  return pl.pallas_call(
      _tc_body,
      grid=(B, G // XB, YZ // TYZ),
      in_specs=[in_spec(4), in_spec(80), in_spec(4), in_spec(80),
                in_spec(2 * D)],
      out_specs=[out_spec, out_spec],
      out_shape=[jax.ShapeDtypeStruct((B, YZ, G, nc), jnp.float32),
                 jax.ShapeDtypeStruct((B, YZ, G, nc), jnp.float32)],
  )(aj, pj, af, pf, gc)


def kernel(voxels_allAtom_jigsaw, voxels_perAA_jigsaw, voxels_allAtom_full,
           voxels_perAA_full, prot_feats, centerIdx_jigsaw, resIds_jigsaw,
           centerIdx_full, resIds_full):
  z = jnp.zeros((NRES + 1, D), jnp.float32)
  pp = z.at[:NRES].set(prot_feats)
  protD = jnp.concatenate([pp, pp], axis=1)
  protZL = jnp.concatenate([jnp.zeros_like(pp), pp], axis=1)
  protZR = jnp.concatenate([pp, jnp.zeros_like(pp)], axis=1)
  gc = _sc_scatter(protD, protZL, protZR,
                   centerIdx_jigsaw.astype(jnp.int32),
                   resIds_jigsaw.astype(jnp.int32),
                   centerIdx_full.astype(jnp.int32),
                   resIds_full.astype(jnp.int32))
  gc4 = gc.reshape(B, G, G * G, 2 * D)
  aj = voxels_allAtom_jigsaw.reshape(B, G, G * G, 4)
  pj = voxels_perAA_jigsaw.reshape(B, G, G * G, 80)
  af = voxels_allAtom_full.reshape(B, G, G * G, 4)
  pf = voxels_perAA_full.reshape(B, G, G * G, 80)
  oj, of = _tc_fuse(aj, pj, af, pf, gc4)
  oj = jnp.transpose(oj.reshape(B, G, G, G, 150), (0, 4, 1, 2, 3))
  of = jnp.transpose(of.reshape(B, G, G, G, 150), (0, 4, 1, 2, 3))
  return (oj, of)
